# baseline (speedup 1.0000x reference)
import functools

import jax
import jax.numpy as jnp
from jax import lax
from jax.experimental import pallas as pl
from jax.experimental.pallas import tpu as pltpu
from jax.experimental.pallas import tpu_sc as plsc

N = 10000
E = 320000
D_IN = 128
DH = 256
HALF = 128
D_OUT = 128
G = 16

TILES = 16
CH = 64
NB = 10
B = 32
NCH = NB * B
EP = NCH * CH
PAD = TILES * EP - E
JUNK = N

ACC_W = 10112
ZPT = ACC_W // TILES
OUT_ROWS = ACC_W

R = 1000


def _agg_body(hsa, hsb, srcs, dsts, zrows, outa, outb,
              src_v, dst_v, buf, acc_s, sem):
    c = lax.axis_index("c")
    s = lax.axis_index("s")
    pltpu.sync_copy(zrows, acc_s.at[pl.ds(s * ZPT, ZPT)])
    plsc.subcore_barrier()

    def run(hs, out):
        def blk(nb, carry):
            pltpu.sync_copy(srcs.at[s * NB + nb], src_v)
            pltpu.sync_copy(dsts.at[s * NB + nb], dst_v)

            def body(j, carry2):
                idx = src_v.at[pl.ds(j * CH, CH)]
                pltpu.async_copy(hs.at[idx], buf, sem).wait()
                pltpu.sync_copy(buf, acc_s.at[dst_v.at[j]], add=True)
                return carry2

            lax.fori_loop(0, B, body, 0)
            return carry

        lax.fori_loop(0, NB, blk, 0)
        plsc.subcore_barrier()
        lo = s * ZPT
        pltpu.sync_copy(acc_s.at[pl.ds(lo, ZPT)], out.at[pl.ds(lo, ZPT)])

    @pl.when(c == 0)
    def _():
        run(hsa, outa)

    @pl.when(c == 1)
    def _():
        run(hsb, outb)


@functools.lru_cache(maxsize=None)
def _sc_kernels():
    mesh = plsc.VectorSubcoreMesh(core_axis_name="c", subcore_axis_name="s")
    agg_call = pl.kernel(
        _agg_body,
        mesh=mesh,
        out_type=[
            jax.ShapeDtypeStruct((OUT_ROWS, HALF), jnp.float32),
            jax.ShapeDtypeStruct((OUT_ROWS, HALF), jnp.float32),
        ],
        scratch_types=[
            pltpu.VMEM((B * CH,), jnp.int32),
            pltpu.VMEM((B, CH), jnp.int32),
            pltpu.VMEM((CH, HALF), jnp.float32),
            pltpu.VMEM_SHARED((ACC_W, HALF), jnp.float32),
            pltpu.SemaphoreType.DMA,
        ],
    )
    return agg_call


def _dinv_of(dego):
    return lax.rsqrt(dego[:, 0:1] + 1.0)


def _tc_a_body(x_ref, w1_ref, dego_ref, hsa_ref, hsb_ref):
    h = jnp.dot(x_ref[...], w1_ref[...], preferred_element_type=jnp.float32)
    dinv = _dinv_of(dego_ref[...])
    hs = h * dinv
    hsa_ref[...] = hs[:, :HALF]
    hsb_ref[...] = hs[:, HALF:]


def _tc_a(x, W1, dego):
    return pl.pallas_call(
        _tc_a_body,
        grid=(N // R,),
        in_specs=[
            pl.BlockSpec((R, D_IN), lambda i: (i, 0)),
            pl.BlockSpec((D_IN, DH), lambda i: (0, 0)),
            pl.BlockSpec((R, HALF), lambda i: (i, 0)),
        ],
        out_specs=[
            pl.BlockSpec((R, HALF), lambda i: (i, 0)),
            pl.BlockSpec((R, HALF), lambda i: (i, 0)),
        ],
        out_shape=[
            jax.ShapeDtypeStruct((N, HALF), jnp.float32),
            jax.ShapeDtypeStruct((N, HALF), jnp.float32),
        ],
    )(x, W1, dego)


def _post_agg(acca, accb, hsa, hsb, dinv, b, g, be):
    acc = jnp.concatenate([acca, accb], axis=1)
    hs = jnp.concatenate([hsa, hsb], axis=1)
    t = dinv * (acc + hs) + b
    r = jnp.maximum(t, 0.0)
    mu = jnp.mean(r, axis=1, keepdims=True)
    var = jnp.mean((r - mu) * (r - mu), axis=1, keepdims=True)
    return (r - mu) * lax.rsqrt(var + 1e-5) * g + be


def _tc_b_body(acca_ref, accb_ref, hsa_ref, hsb_ref, dego_ref,
               b_ref, g_ref, be_ref, w_ref, outa_ref, outb_ref):
    dinv = _dinv_of(dego_ref[...])
    hn = _post_agg(acca_ref[...], accb_ref[...], hsa_ref[...], hsb_ref[...],
                   dinv, b_ref[...], g_ref[...], be_ref[...])
    hs2 = jnp.dot(hn, w_ref[...], preferred_element_type=jnp.float32) * dinv
    outa_ref[...] = hs2[:, :HALF]
    outb_ref[...] = hs2[:, HALF:]


def _tc_b(acca, accb, hsa, hsb, dego, b1, g1, be1, W2):
    row = pl.BlockSpec((R, HALF), lambda i: (i, 0))
    vec = pl.BlockSpec((1, DH), lambda i: (0, 0))
    return pl.pallas_call(
        _tc_b_body,
        grid=(N // R,),
        in_specs=[
            row, row, row, row,
            row,
            vec, vec, vec,
            pl.BlockSpec((DH, DH), lambda i: (0, 0)),
        ],
        out_specs=[row, row],
        out_shape=[
            jax.ShapeDtypeStruct((N, HALF), jnp.float32),
            jax.ShapeDtypeStruct((N, HALF), jnp.float32),
        ],
    )(acca, accb, hsa, hsb, dego, b1, g1, be1, W2)


def _tc_c_body(acca_ref, accb_ref, hsa_ref, hsb_ref, dego_ref,
               b_ref, g_ref, be_ref, batch_ref,
               fcw1_ref, fcb1_ref, fcw2_ref, fcb2_ref,
               out_ref, sums_s, cnt_s, max_s):
    i = pl.program_id(0)
    dinv = _dinv_of(dego_ref[...])
    hn = _post_agg(acca_ref[...], accb_ref[...], hsa_ref[...], hsb_ref[...],
                   dinv, b_ref[...], g_ref[...], be_ref[...])

    @pl.when(i == 0)
    def _():
        sums_s[...] = jnp.zeros_like(sums_s)
        cnt_s[...] = jnp.zeros_like(cnt_s)
        max_s[...] = jnp.full_like(max_s, -jnp.inf)

    b = batch_ref[...]
    onehot = (b == lax.broadcasted_iota(jnp.int32, (1, G), 1)).astype(jnp.float32)
    dims = (((0,), (0,)), ((), ()))
    sums_s[...] += lax.dot_general(onehot, hn, dims,
                                   preferred_element_type=jnp.float32)
    cnt_s[...] += lax.dot_general(onehot, jnp.ones_like(hn), dims,
                                  preferred_element_type=jnp.float32)
    rows = []
    for g in range(G):
        m = (b == g)
        rows.append(jnp.max(jnp.where(m, hn, -jnp.inf), axis=0, keepdims=True))
    max_s[...] = jnp.maximum(max_s[...], jnp.concatenate(rows, axis=0))

    @pl.when(i == (N // R) - 1)
    def _():
        sums = sums_s[...]
        cnt = jnp.maximum(cnt_s[...], 1.0)
        pooled = jnp.concatenate([sums / cnt, sums, max_s[...]], axis=1)
        o1 = jnp.dot(pooled, fcw1_ref[...], preferred_element_type=jnp.float32)
        o1 = jnp.maximum(o1 + fcb1_ref[...], 0.0)
        out_ref[...] = (jnp.dot(o1, fcw2_ref[...],
                                preferred_element_type=jnp.float32)
                        + fcb2_ref[...])


def _tc_c(acca, accb, hsa, hsb, dego, b2, g2, be2, batch2,
          fcW1, fcb1, fcW2, fcb2):
    row = pl.BlockSpec((R, HALF), lambda i: (i, 0))
    vec = pl.BlockSpec((1, DH), lambda i: (0, 0))
    return pl.pallas_call(
        _tc_c_body,
        grid=(N // R,),
        in_specs=[
            row, row, row, row,
            row,
            vec, vec, vec,
            pl.BlockSpec((R, 1), lambda i: (i, 0)),
            pl.BlockSpec((3 * DH, DH), lambda i: (0, 0)),
            vec,
            pl.BlockSpec((DH, D_OUT), lambda i: (0, 0)),
            pl.BlockSpec((1, D_OUT), lambda i: (0, 0)),
        ],
        out_specs=pl.BlockSpec((G, D_OUT), lambda i: (0, 0)),
        out_shape=jax.ShapeDtypeStruct((G, D_OUT), jnp.float32),
        scratch_shapes=[
            pltpu.VMEM((G, DH), jnp.float32),
            pltpu.VMEM((G, DH), jnp.float32),
            pltpu.VMEM((G, DH), jnp.float32),
        ],
    )(acca, accb, hsa, hsb, dego, b2, g2, be2, batch2,
      fcW1, fcb1, fcW2, fcb2)


def kernel(x, edge_index, batch, W1, b1, W2, b2, g1, be1, g2, be2,
           fcW1, fcb1, fcW2, fcb2):
    src = edge_index[0].astype(jnp.int32)
    dst = edge_index[1].astype(jnp.int32)
    srcs = jnp.concatenate([src, jnp.zeros((PAD,), jnp.int32)])
    srcs = srcs.reshape(TILES * NB, B * CH)
    dstp = jnp.concatenate([dst, jnp.full((PAD,), JUNK, jnp.int32)])
    dsts = dstp.reshape(TILES * NB, B, CH)
    zrows = jnp.zeros((ZPT, HALF), jnp.float32)
    onesh = jnp.ones((N, HALF), jnp.float32)
    batch2 = batch.astype(jnp.int32).reshape(N, 1)

    agg_call = _sc_kernels()
    dego, _unused = agg_call(onesh, onesh, srcs, dsts, zrows)
    hs1a, hs1b = _tc_a(x, W1, dego)
    acc1a, acc1b = agg_call(hs1a, hs1b, srcs, dsts, zrows)
    hs2a, hs2b = _tc_b(acc1a, acc1b, hs1a, hs1b, dego,
                       b1.reshape(1, -1), g1.reshape(1, -1),
                       be1.reshape(1, -1), W2)
    acc2a, acc2b = agg_call(hs2a, hs2b, srcs, dsts, zrows)
    return _tc_c(acc2a, acc2b, hs2a, hs2b, dego,
                 b2.reshape(1, -1), g2.reshape(1, -1), be2.reshape(1, -1),
                 batch2, fcW1, fcb1.reshape(1, -1), fcW2, fcb2.reshape(1, -1))


# speedup vs baseline: 1.2377x; 1.2377x over previous
import functools

import jax
import jax.numpy as jnp
from jax import lax
from jax.experimental import pallas as pl
from jax.experimental.pallas import tpu as pltpu
from jax.experimental.pallas import tpu_sc as plsc

N = 10000
E = 320000
D_IN = 128
DH = 256
HALF = 128
D_OUT = 128
G = 16

TILES = 16
CH = 64
NB = 10
B = 32
NCH = NB * B
EP = NCH * CH
PAD = TILES * EP - E
JUNK = N

ACC_W = 10112
ZPT = ACC_W // TILES
OUT_ROWS = ACC_W

R = 1000


def _agg_body(hsa, hsb, srcs, dsts, zrows, outa, outb,
              src_v, dst_v, buf0, buf1, acc_s, sem0, sem1):
    c = lax.axis_index("c")
    s = lax.axis_index("s")
    pltpu.sync_copy(zrows, acc_s.at[pl.ds(s * ZPT, ZPT)])
    plsc.subcore_barrier()

    def run(hs, out):
        def blk(nb, carry):
            pltpu.sync_copy(srcs.at[s * NB + nb], src_v)
            pltpu.sync_copy(dsts.at[s * NB + nb], dst_v)
            pltpu.make_async_copy(
                hs.at[src_v.at[pl.ds(0, CH)]], buf0, sem0).start()
            pltpu.make_async_copy(
                hs.at[src_v.at[pl.ds(CH, CH)]], buf1, sem1).start()

            def grp(g, carry2):
                for k, bf, sm in ((0, buf0, sem0), (1, buf1, sem1)):
                    j = 2 * g + k
                    pltpu.make_async_copy(
                        hs.at[src_v.at[pl.ds(0, CH)]], bf, sm).wait()
                    pltpu.sync_copy(bf, acc_s.at[dst_v.at[j]], add=True)

                    @pl.when(g < B // 2 - 1)
                    def _():
                        idx = src_v.at[pl.ds((j + 2) * CH, CH)]
                        pltpu.make_async_copy(hs.at[idx], bf, sm).start()
                return carry2

            lax.fori_loop(0, B // 2, grp, 0)
            return carry

        lax.fori_loop(0, NB, blk, 0)
        plsc.subcore_barrier()
        lo = s * ZPT
        pltpu.sync_copy(acc_s.at[pl.ds(lo, ZPT)], out.at[pl.ds(lo, ZPT)])

    @pl.when(c == 0)
    def _():
        run(hsa, outa)

    @pl.when(c == 1)
    def _():
        run(hsb, outb)


@functools.lru_cache(maxsize=None)
def _sc_kernels():
    mesh = plsc.VectorSubcoreMesh(core_axis_name="c", subcore_axis_name="s")
    agg_call = pl.kernel(
        _agg_body,
        mesh=mesh,
        out_type=[
            jax.ShapeDtypeStruct((OUT_ROWS, HALF), jnp.float32),
            jax.ShapeDtypeStruct((OUT_ROWS, HALF), jnp.float32),
        ],
        scratch_types=[
            pltpu.VMEM((B * CH,), jnp.int32),
            pltpu.VMEM((B, CH), jnp.int32),
            pltpu.VMEM((CH, HALF), jnp.float32),
            pltpu.VMEM((CH, HALF), jnp.float32),
            pltpu.VMEM_SHARED((ACC_W, HALF), jnp.float32),
            pltpu.SemaphoreType.DMA,
            pltpu.SemaphoreType.DMA,
        ],
    )
    return agg_call


def _dinv_of(dego):
    return lax.rsqrt(dego[:, 0:1] + 1.0)


def _tc_a_body(x_ref, w1_ref, dego_ref, hsa_ref, hsb_ref):
    h = jnp.dot(x_ref[...], w1_ref[...], preferred_element_type=jnp.float32)
    dinv = _dinv_of(dego_ref[...])
    hs = h * dinv
    hsa_ref[...] = hs[:, :HALF]
    hsb_ref[...] = hs[:, HALF:]


def _tc_a(x, W1, dego):
    return pl.pallas_call(
        _tc_a_body,
        grid=(N // R,),
        in_specs=[
            pl.BlockSpec((R, D_IN), lambda i: (i, 0)),
            pl.BlockSpec((D_IN, DH), lambda i: (0, 0)),
            pl.BlockSpec((R, HALF), lambda i: (i, 0)),
        ],
        out_specs=[
            pl.BlockSpec((R, HALF), lambda i: (i, 0)),
            pl.BlockSpec((R, HALF), lambda i: (i, 0)),
        ],
        out_shape=[
            jax.ShapeDtypeStruct((N, HALF), jnp.float32),
            jax.ShapeDtypeStruct((N, HALF), jnp.float32),
        ],
    )(x, W1, dego)


def _post_agg(acca, accb, hsa, hsb, dinv, b, g, be):
    acc = jnp.concatenate([acca, accb], axis=1)
    hs = jnp.concatenate([hsa, hsb], axis=1)
    t = dinv * (acc + hs) + b
    r = jnp.maximum(t, 0.0)
    mu = jnp.mean(r, axis=1, keepdims=True)
    var = jnp.mean((r - mu) * (r - mu), axis=1, keepdims=True)
    return (r - mu) * lax.rsqrt(var + 1e-5) * g + be


def _tc_b_body(acca_ref, accb_ref, hsa_ref, hsb_ref, dego_ref,
               b_ref, g_ref, be_ref, w_ref, outa_ref, outb_ref):
    dinv = _dinv_of(dego_ref[...])
    hn = _post_agg(acca_ref[...], accb_ref[...], hsa_ref[...], hsb_ref[...],
                   dinv, b_ref[...], g_ref[...], be_ref[...])
    hs2 = jnp.dot(hn, w_ref[...], preferred_element_type=jnp.float32) * dinv
    outa_ref[...] = hs2[:, :HALF]
    outb_ref[...] = hs2[:, HALF:]


def _tc_b(acca, accb, hsa, hsb, dego, b1, g1, be1, W2):
    row = pl.BlockSpec((R, HALF), lambda i: (i, 0))
    vec = pl.BlockSpec((1, DH), lambda i: (0, 0))
    return pl.pallas_call(
        _tc_b_body,
        grid=(N // R,),
        in_specs=[
            row, row, row, row,
            row,
            vec, vec, vec,
            pl.BlockSpec((DH, DH), lambda i: (0, 0)),
        ],
        out_specs=[row, row],
        out_shape=[
            jax.ShapeDtypeStruct((N, HALF), jnp.float32),
            jax.ShapeDtypeStruct((N, HALF), jnp.float32),
        ],
    )(acca, accb, hsa, hsb, dego, b1, g1, be1, W2)


def _tc_c_body(acca_ref, accb_ref, hsa_ref, hsb_ref, dego_ref,
               b_ref, g_ref, be_ref, batch_ref,
               fcw1_ref, fcb1_ref, fcw2_ref, fcb2_ref,
               out_ref, sums_s, cnt_s, max_s):
    i = pl.program_id(0)
    dinv = _dinv_of(dego_ref[...])
    hn = _post_agg(acca_ref[...], accb_ref[...], hsa_ref[...], hsb_ref[...],
                   dinv, b_ref[...], g_ref[...], be_ref[...])

    @pl.when(i == 0)
    def _():
        sums_s[...] = jnp.zeros_like(sums_s)
        cnt_s[...] = jnp.zeros_like(cnt_s)
        max_s[...] = jnp.full_like(max_s, -jnp.inf)

    b = batch_ref[...]
    onehot = (b == lax.broadcasted_iota(jnp.int32, (1, G), 1)).astype(jnp.float32)
    dims = (((0,), (0,)), ((), ()))
    sums_s[...] += lax.dot_general(onehot, hn, dims,
                                   preferred_element_type=jnp.float32)
    cnt_s[...] += lax.dot_general(onehot, jnp.ones_like(hn), dims,
                                  preferred_element_type=jnp.float32)
    rows = []
    for g in range(G):
        m = (b == g)
        rows.append(jnp.max(jnp.where(m, hn, -jnp.inf), axis=0, keepdims=True))
    max_s[...] = jnp.maximum(max_s[...], jnp.concatenate(rows, axis=0))

    @pl.when(i == (N // R) - 1)
    def _():
        sums = sums_s[...]
        cnt = jnp.maximum(cnt_s[...], 1.0)
        pooled = jnp.concatenate([sums / cnt, sums, max_s[...]], axis=1)
        o1 = jnp.dot(pooled, fcw1_ref[...], preferred_element_type=jnp.float32)
        o1 = jnp.maximum(o1 + fcb1_ref[...], 0.0)
        out_ref[...] = (jnp.dot(o1, fcw2_ref[...],
                                preferred_element_type=jnp.float32)
                        + fcb2_ref[...])


def _tc_c(acca, accb, hsa, hsb, dego, b2, g2, be2, batch2,
          fcW1, fcb1, fcW2, fcb2):
    row = pl.BlockSpec((R, HALF), lambda i: (i, 0))
    vec = pl.BlockSpec((1, DH), lambda i: (0, 0))
    return pl.pallas_call(
        _tc_c_body,
        grid=(N // R,),
        in_specs=[
            row, row, row, row,
            row,
            vec, vec, vec,
            pl.BlockSpec((R, 1), lambda i: (i, 0)),
            pl.BlockSpec((3 * DH, DH), lambda i: (0, 0)),
            vec,
            pl.BlockSpec((DH, D_OUT), lambda i: (0, 0)),
            pl.BlockSpec((1, D_OUT), lambda i: (0, 0)),
        ],
        out_specs=pl.BlockSpec((G, D_OUT), lambda i: (0, 0)),
        out_shape=jax.ShapeDtypeStruct((G, D_OUT), jnp.float32),
        scratch_shapes=[
            pltpu.VMEM((G, DH), jnp.float32),
            pltpu.VMEM((G, DH), jnp.float32),
            pltpu.VMEM((G, DH), jnp.float32),
        ],
    )(acca, accb, hsa, hsb, dego, b2, g2, be2, batch2,
      fcW1, fcb1, fcW2, fcb2)


def kernel(x, edge_index, batch, W1, b1, W2, b2, g1, be1, g2, be2,
           fcW1, fcb1, fcW2, fcb2):
    src = edge_index[0].astype(jnp.int32)
    dst = edge_index[1].astype(jnp.int32)
    srcs = jnp.concatenate([src, jnp.zeros((PAD,), jnp.int32)])
    srcs = srcs.reshape(TILES * NB, B * CH)
    dstp = jnp.concatenate([dst, jnp.full((PAD,), JUNK, jnp.int32)])
    dsts = dstp.reshape(TILES * NB, B, CH)
    zrows = jnp.zeros((ZPT, HALF), jnp.float32)
    onesh = jnp.ones((N, HALF), jnp.float32)
    batch2 = batch.astype(jnp.int32).reshape(N, 1)

    agg_call = _sc_kernels()
    dego, _unused = agg_call(onesh, onesh, srcs, dsts, zrows)
    hs1a, hs1b = _tc_a(x, W1, dego)
    acc1a, acc1b = agg_call(hs1a, hs1b, srcs, dsts, zrows)
    hs2a, hs2b = _tc_b(acc1a, acc1b, hs1a, hs1b, dego,
                       b1.reshape(1, -1), g1.reshape(1, -1),
                       be1.reshape(1, -1), W2)
    acc2a, acc2b = agg_call(hs2a, hs2b, srcs, dsts, zrows)
    return _tc_c(acc2a, acc2b, hs2a, hs2b, dego,
                 b2.reshape(1, -1), g2.reshape(1, -1), be2.reshape(1, -1),
                 batch2, fcW1, fcb1.reshape(1, -1), fcW2, fcb2.reshape(1, -1))
